# baseline (speedup 1.0000x reference)
import jax
import jax.numpy as jnp
from jax.experimental import pallas as pl
from jax.experimental.pallas import tpu as pltpu

PS = 5
K = 14
PAD = PS // 2
BQ = 1024
NQB = 4096 // BQ


def _body(misc_ref, gps_ref, gpsT_ref, cps_ref, W1_ref, b1_ref,
          Wcat_ref, bcat_ref, out_ref, cw_scr, imr_scr, pw_scr):
    t = pl.program_id(0)
    qb = pl.program_id(1)
    del t

    @pl.when(qb == 0)
    def _init():
        cw_scr[...] = jnp.dot(cps_ref[0], W1_ref[...],
                              preferred_element_type=jnp.float32)

    gps = gps_ref[0]
    gqT = gpsT_ref[0]
    sq_c = jnp.sum(gps * gps, axis=1, keepdims=True)
    sq_q = jnp.sum(gqT * gqT, axis=0, keepdims=True)
    mm = jax.lax.dot_general(gps, gqT, (((1,), (0,)), ((), ())),
                             preferred_element_type=jnp.float32)
    dist = sq_c + sq_q - 2.0 * mm

    big = jnp.float32(3.0e38)
    NCH, CH, S = 64, 4096 // 64, 5
    D3 = dist.reshape(NCH, CH, BQ)
    p = jnp.min(D3, axis=1)
    pool = [p]
    for _ in range(S - 1):
        p = jnp.min(jnp.where(D3 > p[:, None, :], D3, big), axis=1)
        pool.append(p)
    P = jnp.concatenate(pool, axis=0)
    g = jnp.min(P, axis=0, keepdims=True)
    gs = [g]
    for _ in range(K - 1):
        g = jnp.min(jnp.where(P > g, P, big), axis=0, keepdims=True)
        gs.append(g)
    G = jnp.concatenate(gs, axis=0)
    ok = jnp.min(jnp.where(pool[-1] > G[K - 1:K], 1.0, 0.0))

    def _full_chain():
        m = jnp.min(dist, axis=0, keepdims=True)
        ms = [m]
        for _ in range(K - 1):
            m = jnp.min(jnp.where(dist > m, dist, big),
                        axis=0, keepdims=True)
            ms.append(m)
        return jnp.concatenate(ms, axis=0)

    msall = jax.lax.cond(ok > 0.5, lambda: G, _full_chain)
    m1 = msall[0:1]
    thresh = msall[K - 1:K]

    T = misc_ref[0, 0, 0]
    invT = 1.0 / T
    w = jnp.where(dist <= thresh, jnp.exp((m1 - dist) * invT), 0.0)
    Z = jnp.sum(jnp.exp((m1 - msall) * invT), axis=0, keepdims=True)

    accT = jax.lax.dot_general(cw_scr[...], w, (((0,), (0,)), ((), ())),
                               preferred_element_type=jnp.float32)
    featT = jnp.maximum(accT / Z + b1_ref[...], 0.0)
    imrT = jax.lax.dot_general(Wcat_ref[...], featT, (((0,), (0,)), ((), ())),
                               preferred_element_type=jnp.float32)
    imrT = imrT + bcat_ref[...]
    pw = jax.nn.sigmoid(imrT[75:76, :])
    imr_scr[:, pl.ds(qb * BQ, BQ)] = imrT[0:75, :] * pw
    pw_scr[:, pl.ds(qb * BQ, BQ)] = pw

    @pl.when(qb == NQB - 1)
    def _fold():
        Q = 4096
        colid = jax.lax.broadcasted_iota(jnp.int32, (1, Q), 1) & 63
        zpad = jnp.zeros((1, 192), jnp.float32)

        def shifted(row, s):
            ext = jnp.concatenate([zpad, row, zpad], axis=1)
            return jax.lax.slice(ext, (0, 192 - s), (1, 192 - s + Q))

        den = jnp.zeros((1, Q), jnp.float32)
        pwrow = pw_scr[0:1, :]
        masks = {}
        for a in range(-2, 3):
            for b in range(-2, 3):
                mask = jnp.logical_and(colid - b >= 0, colid - b < 64)
                masks[(a, b)] = mask
                den = den + jnp.where(mask, shifted(pwrow, a * 64 + b), 0.0)
        rden = 1.0 / (den + 1e-10)
        for c in range(3):
            num = jnp.zeros((1, Q), jnp.float32)
            for a in range(-2, 3):
                for b in range(-2, 3):
                    f = c * 25 + (a + 2) * 5 + (b + 2)
                    row = imr_scr[f:f + 1, :]
                    num = num + jnp.where(masks[(a, b)],
                                          shifted(row, a * 64 + b), 0.0)
            mean_c = misc_ref[0, 0, 1 + c]
            out_ref[0, c, :] = 127.5 * ((num[0] * rden[0]) + mean_c) + 127.5


def kernel(noisy, sigma, W1, b1, W2, b2, Wg, bg):
    t, c, h, w = noisy.shape
    pdim = c * PS * PS
    x = (noisy / 255.0 - 0.5) / 0.5
    means = x.mean(axis=(-2, -1), keepdims=True)
    x = x - means
    gw = jnp.array([0.2989, 0.587, 0.114], jnp.float32).reshape(1, 3, 1, 1)
    gray = (x * gw).sum(axis=1)
    grayp = jnp.pad(gray, ((0, 0), (PAD, PAD), (PAD, PAD)), mode='reflect')
    xp = jnp.pad(x, ((0, 0), (0, 0), (PAD, PAD), (PAD, PAD)), mode='reflect')
    gps = jnp.stack([grayp[:, i:i + h, j:j + w]
                     for i in range(PS) for j in range(PS)], axis=-1)
    gps = gps.reshape(t, h * w, PS * PS)
    gps32 = jnp.pad(gps, ((0, 0), (0, 0), (0, 7)))
    gpsT = jnp.transpose(gps32, (0, 2, 1))
    cps = jnp.stack([xp[:, :, i:i + h, j:j + w]
                     for i in range(PS) for j in range(PS)], axis=-1)
    cps = cps.transpose(0, 2, 3, 1, 4).reshape(t, h * w, pdim)

    sig = sigma[0] / 255.0
    T = pdim * sig * sig + 1e-6
    means_tc = means.reshape(t, c)
    misc = jnp.concatenate(
        [jnp.broadcast_to(T.reshape(1, 1), (t, 1)), means_tc,
         jnp.zeros((t, 4), jnp.float32)], axis=1).reshape(t, 1, 8)

    Wcat = jnp.concatenate([W2, Wg], axis=1)
    bcat = jnp.concatenate([b2, bg]).reshape(76, 1)
    b1r = b1.reshape(80, 1)

    grid = (t, NQB)
    out = pl.pallas_call(
        _body,
        grid=grid,
        in_specs=[
            pl.BlockSpec((1, 1, 8), lambda ti, qb: (ti, 0, 0)),
            pl.BlockSpec((1, h * w, 32), lambda ti, qb: (ti, 0, 0)),
            pl.BlockSpec((1, 32, BQ), lambda ti, qb: (ti, 0, qb)),
            pl.BlockSpec((1, h * w, pdim), lambda ti, qb: (ti, 0, 0)),
            pl.BlockSpec((pdim, 80), lambda ti, qb: (0, 0)),
            pl.BlockSpec((80, 1), lambda ti, qb: (0, 0)),
            pl.BlockSpec((80, 76), lambda ti, qb: (0, 0)),
            pl.BlockSpec((76, 1), lambda ti, qb: (0, 0)),
        ],
        out_specs=pl.BlockSpec((1, 3, h * w), lambda ti, qb: (ti, 0, 0)),
        out_shape=jax.ShapeDtypeStruct((t, 3, h * w), jnp.float32),
        scratch_shapes=[
            pltpu.VMEM((h * w, 80), jnp.float32),
            pltpu.VMEM((75, h * w), jnp.float32),
            pltpu.VMEM((1, h * w), jnp.float32),
        ],
    )(misc, gps32, gpsT, cps, W1, b1r, Wcat, bcat)
    return out.reshape(t, 3, h, w)


# speedup vs baseline: 1.7698x; 1.7698x over previous
import jax
import jax.numpy as jnp
from jax.experimental import pallas as pl
from jax.experimental.pallas import tpu as pltpu

PS = 5
K = 14
PAD = PS // 2
BQ = 256
NQB = 4096 // BQ


def _body(misc_ref, gps_ref, gpsT_ref, cps_ref, W1_ref, b1_ref,
          Wcat_ref, bcat_ref, out_ref, cw_scr, imr_scr, pw_scr):
    t = pl.program_id(0)
    qb = pl.program_id(1)
    del t

    @pl.when(qb == 0)
    def _init():
        cw_scr[...] = jnp.dot(cps_ref[0], W1_ref[...],
                              preferred_element_type=jnp.float32)

    gps = gps_ref[0]
    gqT = gpsT_ref[0]
    sq_c = jnp.sum(gps * gps, axis=1, keepdims=True)
    sq_q = jnp.sum(gqT * gqT, axis=0, keepdims=True)
    mm = jax.lax.dot_general(gps, gqT, (((1,), (0,)), ((), ())),
                             preferred_element_type=jnp.float32)
    dist = sq_c + sq_q - 2.0 * mm

    big = jnp.float32(3.0e38)
    NCH, CH, S = 64, 4096 // 64, 5
    D3 = dist.reshape(NCH, CH, BQ)
    p = jnp.min(D3, axis=1)
    pool = [p]
    for _ in range(S - 1):
        p = jnp.min(jnp.where(D3 > p[:, None, :], D3, big), axis=1)
        pool.append(p)
    P = jnp.concatenate(pool, axis=0)
    g = jnp.min(P, axis=0, keepdims=True)
    gs = [g]
    for _ in range(K - 1):
        g = jnp.min(jnp.where(P > g, P, big), axis=0, keepdims=True)
        gs.append(g)
    G = jnp.concatenate(gs, axis=0)
    ok = jnp.min(jnp.where(pool[-1] > G[K - 1:K], 1.0, 0.0))

    def _full_chain():
        m = jnp.min(dist, axis=0, keepdims=True)
        ms = [m]
        for _ in range(K - 1):
            m = jnp.min(jnp.where(dist > m, dist, big),
                        axis=0, keepdims=True)
            ms.append(m)
        return jnp.concatenate(ms, axis=0)

    msall = jax.lax.cond(ok > 0.5, lambda: G, _full_chain)
    m1 = msall[0:1]
    thresh = msall[K - 1:K]

    T = misc_ref[0, 0, 0]
    invT = 1.0 / T
    w = jnp.where(dist <= thresh, jnp.exp((m1 - dist) * invT), 0.0)
    Z = jnp.sum(jnp.exp((m1 - msall) * invT), axis=0, keepdims=True)

    accT = jax.lax.dot_general(cw_scr[...], w, (((0,), (0,)), ((), ())),
                               preferred_element_type=jnp.float32)
    featT = jnp.maximum(accT / Z + b1_ref[...], 0.0)
    imrT = jax.lax.dot_general(Wcat_ref[...], featT, (((0,), (0,)), ((), ())),
                               preferred_element_type=jnp.float32)
    imrT = imrT + bcat_ref[...]
    pw = jax.nn.sigmoid(imrT[75:76, :])
    imr_scr[:, pl.ds(qb * BQ, BQ)] = imrT[0:75, :] * pw
    pw_scr[:, pl.ds(qb * BQ, BQ)] = pw

    @pl.when(qb == NQB - 1)
    def _fold():
        Q = 4096
        colid = jax.lax.broadcasted_iota(jnp.int32, (1, Q), 1) & 63
        zpad = jnp.zeros((1, 192), jnp.float32)

        def shifted(row, s):
            ext = jnp.concatenate([zpad, row, zpad], axis=1)
            return jax.lax.slice(ext, (0, 192 - s), (1, 192 - s + Q))

        den = jnp.zeros((1, Q), jnp.float32)
        pwrow = pw_scr[0:1, :]
        masks = {}
        for a in range(-2, 3):
            for b in range(-2, 3):
                mask = jnp.logical_and(colid - b >= 0, colid - b < 64)
                masks[(a, b)] = mask
                den = den + jnp.where(mask, shifted(pwrow, a * 64 + b), 0.0)
        rden = 1.0 / (den + 1e-10)
        for c in range(3):
            num = jnp.zeros((1, Q), jnp.float32)
            for a in range(-2, 3):
                for b in range(-2, 3):
                    f = c * 25 + (a + 2) * 5 + (b + 2)
                    row = imr_scr[f:f + 1, :]
                    num = num + jnp.where(masks[(a, b)],
                                          shifted(row, a * 64 + b), 0.0)
            mean_c = misc_ref[0, 0, 1 + c]
            out_ref[0, c, :] = 127.5 * ((num[0] * rden[0]) + mean_c) + 127.5


def kernel(noisy, sigma, W1, b1, W2, b2, Wg, bg):
    t, c, h, w = noisy.shape
    pdim = c * PS * PS
    x = (noisy / 255.0 - 0.5) / 0.5
    means = x.mean(axis=(-2, -1), keepdims=True)
    x = x - means
    gw = jnp.array([0.2989, 0.587, 0.114], jnp.float32).reshape(1, 3, 1, 1)
    gray = (x * gw).sum(axis=1)
    grayp = jnp.pad(gray, ((0, 0), (PAD, PAD), (PAD, PAD)), mode='reflect')
    xp = jnp.pad(x, ((0, 0), (0, 0), (PAD, PAD), (PAD, PAD)), mode='reflect')
    gps = jnp.stack([grayp[:, i:i + h, j:j + w]
                     for i in range(PS) for j in range(PS)], axis=-1)
    gps = gps.reshape(t, h * w, PS * PS)
    gps32 = jnp.pad(gps, ((0, 0), (0, 0), (0, 7)))
    gpsT = jnp.transpose(gps32, (0, 2, 1))
    cps = jnp.stack([xp[:, :, i:i + h, j:j + w]
                     for i in range(PS) for j in range(PS)], axis=-1)
    cps = cps.transpose(0, 2, 3, 1, 4).reshape(t, h * w, pdim)

    sig = sigma[0] / 255.0
    T = pdim * sig * sig + 1e-6
    means_tc = means.reshape(t, c)
    misc = jnp.concatenate(
        [jnp.broadcast_to(T.reshape(1, 1), (t, 1)), means_tc,
         jnp.zeros((t, 4), jnp.float32)], axis=1).reshape(t, 1, 8)

    Wcat = jnp.concatenate([W2, Wg], axis=1)
    bcat = jnp.concatenate([b2, bg]).reshape(76, 1)
    b1r = b1.reshape(80, 1)

    grid = (t, NQB)
    out = pl.pallas_call(
        _body,
        grid=grid,
        in_specs=[
            pl.BlockSpec((1, 1, 8), lambda ti, qb: (ti, 0, 0)),
            pl.BlockSpec((1, h * w, 32), lambda ti, qb: (ti, 0, 0)),
            pl.BlockSpec((1, 32, BQ), lambda ti, qb: (ti, 0, qb)),
            pl.BlockSpec((1, h * w, pdim), lambda ti, qb: (ti, 0, 0)),
            pl.BlockSpec((pdim, 80), lambda ti, qb: (0, 0)),
            pl.BlockSpec((80, 1), lambda ti, qb: (0, 0)),
            pl.BlockSpec((80, 76), lambda ti, qb: (0, 0)),
            pl.BlockSpec((76, 1), lambda ti, qb: (0, 0)),
        ],
        out_specs=pl.BlockSpec((1, 3, h * w), lambda ti, qb: (ti, 0, 0)),
        out_shape=jax.ShapeDtypeStruct((t, 3, h * w), jnp.float32),
        scratch_shapes=[
            pltpu.VMEM((h * w, 80), jnp.float32),
            pltpu.VMEM((75, h * w), jnp.float32),
            pltpu.VMEM((1, h * w), jnp.float32),
        ],
    )(misc, gps32, gpsT, cps, W1, b1r, Wcat, bcat)
    return out.reshape(t, 3, h, w)


# speedup vs baseline: 1.8360x; 1.0374x over previous
import jax
import jax.numpy as jnp
from jax.experimental import pallas as pl
from jax.experimental.pallas import tpu as pltpu

PS = 5
K = 14
PAD = PS // 2
BQ = 256
NQB = 4096 // BQ


def _body(misc_ref, gps_ref, gpsT_ref, cps_ref, W1_ref, b1_ref,
          Wcat_ref, bcat_ref, out_ref, cw_scr, imr_scr, pw_scr):
    t = pl.program_id(0)
    qb = pl.program_id(1)
    del t

    @pl.when(qb == 0)
    def _init():
        cw_scr[...] = jnp.dot(cps_ref[0], W1_ref[...],
                              preferred_element_type=jnp.float32)
        imr_scr[:, 0:256] = jnp.zeros((75, 256), jnp.float32)
        imr_scr[:, 256 + 4096:] = jnp.zeros((75, 256), jnp.float32)
        pw_scr[:, 0:256] = jnp.zeros((1, 256), jnp.float32)
        pw_scr[:, 256 + 4096:] = jnp.zeros((1, 256), jnp.float32)

    gps = gps_ref[0]
    gqT = gpsT_ref[0]
    sq_c = jnp.sum(gps * gps, axis=1, keepdims=True)
    sq_q = jnp.sum(gqT * gqT, axis=0, keepdims=True)
    mm = jax.lax.dot_general(gps, gqT, (((1,), (0,)), ((), ())),
                             preferred_element_type=jnp.float32)
    dist = sq_c + sq_q - 2.0 * mm

    big = jnp.float32(3.0e38)
    NCH, CH, S = 64, 4096 // 64, 5
    D3 = dist.reshape(NCH, CH, BQ)
    p = jnp.min(D3, axis=1)
    pool = [p]
    for _ in range(S - 1):
        p = jnp.min(jnp.where(D3 > p[:, None, :], D3, big), axis=1)
        pool.append(p)
    P = jnp.concatenate(pool, axis=0)
    g = jnp.min(P, axis=0, keepdims=True)
    gs = [g]
    for _ in range(K - 1):
        g = jnp.min(jnp.where(P > g, P, big), axis=0, keepdims=True)
        gs.append(g)
    G = jnp.concatenate(gs, axis=0)
    ok = jnp.min(jnp.where(pool[-1] > G[K - 1:K], 1.0, 0.0))

    def _full_chain():
        m = jnp.min(dist, axis=0, keepdims=True)
        ms = [m]
        for _ in range(K - 1):
            m = jnp.min(jnp.where(dist > m, dist, big),
                        axis=0, keepdims=True)
            ms.append(m)
        return jnp.concatenate(ms, axis=0)

    msall = jax.lax.cond(ok > 0.5, lambda: G, _full_chain)
    m1 = msall[0:1]
    thresh = msall[K - 1:K]

    T = misc_ref[0, 0, 0]
    invT = 1.0 / T
    w = jnp.where(dist <= thresh, jnp.exp((m1 - dist) * invT), 0.0)
    Z = jnp.sum(jnp.exp((m1 - msall) * invT), axis=0, keepdims=True)

    accT = jax.lax.dot_general(cw_scr[...], w, (((0,), (0,)), ((), ())),
                               preferred_element_type=jnp.float32)
    featT = jnp.maximum(accT / Z + b1_ref[...], 0.0)
    imrT = jax.lax.dot_general(Wcat_ref[...], featT, (((0,), (0,)), ((), ())),
                               preferred_element_type=jnp.float32)
    imrT = imrT + bcat_ref[...]
    pw = jax.nn.sigmoid(imrT[75:76, :])
    imr_scr[:, pl.ds(256 + qb * BQ, BQ)] = imrT[0:75, :] * pw
    pw_scr[:, pl.ds(256 + qb * BQ, BQ)] = pw

    @pl.when(qb == NQB - 1)
    def _fold():
        Q = 4096
        colid = jax.lax.broadcasted_iota(jnp.int32, (1, Q), 1) & 63

        den = jnp.zeros((1, Q), jnp.float32)
        masks = {}
        for a in range(-2, 3):
            for b in range(-2, 3):
                mask = jnp.logical_and(colid - b >= 0, colid - b < 64)
                masks[(a, b)] = mask
                s = a * 64 + b
                den = den + jnp.where(mask,
                                      pw_scr[0:1, pl.ds(256 - s, Q)], 0.0)
        rden = 1.0 / (den + 1e-10)
        for c in range(3):
            num = jnp.zeros((1, Q), jnp.float32)
            for a in range(-2, 3):
                for b in range(-2, 3):
                    f = c * 25 + (a + 2) * 5 + (b + 2)
                    s = a * 64 + b
                    num = num + jnp.where(masks[(a, b)],
                                          imr_scr[f:f + 1, pl.ds(256 - s, Q)],
                                          0.0)
            mean_c = misc_ref[0, 0, 1 + c]
            out_ref[0, c, :] = 127.5 * ((num[0] * rden[0]) + mean_c) + 127.5


def kernel(noisy, sigma, W1, b1, W2, b2, Wg, bg):
    t, c, h, w = noisy.shape
    pdim = c * PS * PS
    x = (noisy / 255.0 - 0.5) / 0.5
    means = x.mean(axis=(-2, -1), keepdims=True)
    x = x - means
    gw = jnp.array([0.2989, 0.587, 0.114], jnp.float32).reshape(1, 3, 1, 1)
    gray = (x * gw).sum(axis=1)
    grayp = jnp.pad(gray, ((0, 0), (PAD, PAD), (PAD, PAD)), mode='reflect')
    xp = jnp.pad(x, ((0, 0), (0, 0), (PAD, PAD), (PAD, PAD)), mode='reflect')
    gps = jnp.stack([grayp[:, i:i + h, j:j + w]
                     for i in range(PS) for j in range(PS)], axis=-1)
    gps = gps.reshape(t, h * w, PS * PS)
    gps32 = jnp.pad(gps, ((0, 0), (0, 0), (0, 7)))
    gpsT = jnp.transpose(gps32, (0, 2, 1))
    cps = jnp.stack([xp[:, :, i:i + h, j:j + w]
                     for i in range(PS) for j in range(PS)], axis=-1)
    cps = cps.transpose(0, 2, 3, 1, 4).reshape(t, h * w, pdim)

    sig = sigma[0] / 255.0
    T = pdim * sig * sig + 1e-6
    means_tc = means.reshape(t, c)
    misc = jnp.concatenate(
        [jnp.broadcast_to(T.reshape(1, 1), (t, 1)), means_tc,
         jnp.zeros((t, 4), jnp.float32)], axis=1).reshape(t, 1, 8)

    Wcat = jnp.concatenate([W2, Wg], axis=1)
    bcat = jnp.concatenate([b2, bg]).reshape(76, 1)
    b1r = b1.reshape(80, 1)

    grid = (t, NQB)
    out = pl.pallas_call(
        _body,
        grid=grid,
        in_specs=[
            pl.BlockSpec((1, 1, 8), lambda ti, qb: (ti, 0, 0)),
            pl.BlockSpec((1, h * w, 32), lambda ti, qb: (ti, 0, 0)),
            pl.BlockSpec((1, 32, BQ), lambda ti, qb: (ti, 0, qb)),
            pl.BlockSpec((1, h * w, pdim), lambda ti, qb: (ti, 0, 0)),
            pl.BlockSpec((pdim, 80), lambda ti, qb: (0, 0)),
            pl.BlockSpec((80, 1), lambda ti, qb: (0, 0)),
            pl.BlockSpec((80, 76), lambda ti, qb: (0, 0)),
            pl.BlockSpec((76, 1), lambda ti, qb: (0, 0)),
        ],
        out_specs=pl.BlockSpec((1, 3, h * w), lambda ti, qb: (ti, 0, 0)),
        out_shape=jax.ShapeDtypeStruct((t, 3, h * w), jnp.float32),
        scratch_shapes=[
            pltpu.VMEM((h * w, 80), jnp.float32),
            pltpu.VMEM((75, h * w + 512), jnp.float32),
            pltpu.VMEM((1, h * w + 512), jnp.float32),
        ],
    )(misc, gps32, gpsT, cps, W1, b1r, Wcat, bcat)
    return out.reshape(t, 3, h, w)


# speedup vs baseline: 1.8597x; 1.0129x over previous
import jax
import jax.numpy as jnp
from jax.experimental import pallas as pl
from jax.experimental.pallas import tpu as pltpu

PS = 5
K = 14
PAD = PS // 2
BQ = 256
NQB = 4096 // BQ


def _body(misc_ref, g25_ref, cpsT_ref, W1_ref, b1_ref,
          Wcat_ref, bcat_ref, out_ref, cw_scr, A_scr, B_scr,
          imr_scr, pw_scr):
    t = pl.program_id(0)
    qb = pl.program_id(1)
    del t

    @pl.when(qb == 0)
    def _init():
        cw_scr[...] = jax.lax.dot_general(
            cpsT_ref[0], W1_ref[...], (((0,), (0,)), ((), ())),
            preferred_element_type=jnp.float32)
        g = g25_ref[0]
        A_scr[0:25, :] = -2.0 * g
        A_scr[25:26, :] = jnp.sum(g * g, axis=0, keepdims=True)
        A_scr[26:32, :] = jnp.zeros((6, 4096), jnp.float32)
        B_scr[0:25, :] = g
        B_scr[25:26, :] = jnp.ones((1, 4096), jnp.float32)
        B_scr[26:32, :] = jnp.zeros((6, 4096), jnp.float32)
        imr_scr[:, 0:256] = jnp.zeros((75, 256), jnp.float32)
        imr_scr[:, 256 + 4096:] = jnp.zeros((75, 256), jnp.float32)
        pw_scr[:, 0:256] = jnp.zeros((1, 256), jnp.float32)
        pw_scr[:, 256 + 4096:] = jnp.zeros((1, 256), jnp.float32)

    Bq = B_scr[:, pl.ds(qb * BQ, BQ)]
    dist = jax.lax.dot_general(A_scr[...], Bq, (((0,), (0,)), ((), ())),
                               preferred_element_type=jnp.float32)

    big = jnp.float32(3.0e38)
    NCH, CH, S = 64, 4096 // 64, 5
    D3 = dist.reshape(NCH, CH, BQ)
    p = jnp.min(D3, axis=1)
    pool = [p]
    for _ in range(S - 1):
        p = jnp.min(jnp.where(D3 > p[:, None, :], D3, big), axis=1)
        pool.append(p)
    P = jnp.concatenate(pool, axis=0)
    g = jnp.min(P, axis=0, keepdims=True)
    gs = [g]
    for _ in range(K - 1):
        g = jnp.min(jnp.where(P > g, P, big), axis=0, keepdims=True)
        gs.append(g)
    G = jnp.concatenate(gs, axis=0)
    ok = jnp.min(jnp.where(pool[-1] > G[K - 1:K], 1.0, 0.0))

    def _full_chain():
        m = jnp.min(dist, axis=0, keepdims=True)
        ms = [m]
        for _ in range(K - 1):
            m = jnp.min(jnp.where(dist > m, dist, big),
                        axis=0, keepdims=True)
            ms.append(m)
        return jnp.concatenate(ms, axis=0)

    msall = jax.lax.cond(ok > 0.5, lambda: G, _full_chain)
    m1 = msall[0:1]
    thresh = msall[K - 1:K]

    T = misc_ref[0, 0, 0]
    invT = 1.0 / T
    w = jnp.where(dist <= thresh, jnp.exp((m1 - dist) * invT), 0.0)
    Z = jnp.sum(jnp.exp((m1 - msall) * invT), axis=0, keepdims=True)

    accT = jax.lax.dot_general(cw_scr[...], w, (((0,), (0,)), ((), ())),
                               preferred_element_type=jnp.float32)
    featT = jnp.maximum(accT / Z + b1_ref[...], 0.0)
    imrT = jax.lax.dot_general(Wcat_ref[...], featT, (((0,), (0,)), ((), ())),
                               preferred_element_type=jnp.float32)
    imrT = imrT + bcat_ref[...]
    pw = jax.nn.sigmoid(imrT[75:76, :])
    imr_scr[:, pl.ds(256 + qb * BQ, BQ)] = imrT[0:75, :] * pw
    pw_scr[:, pl.ds(256 + qb * BQ, BQ)] = pw

    @pl.when(qb == NQB - 1)
    def _fold():
        Q = 4096
        colid = jax.lax.broadcasted_iota(jnp.int32, (1, Q), 1) & 63

        den = jnp.zeros((1, Q), jnp.float32)
        masks = {}
        for a in range(-2, 3):
            for b in range(-2, 3):
                mask = jnp.logical_and(colid - b >= 0, colid - b < 64)
                masks[(a, b)] = mask
                s = a * 64 + b
                den = den + jnp.where(mask,
                                      pw_scr[0:1, pl.ds(256 - s, Q)], 0.0)
        rden = 1.0 / (den + 1e-10)
        for c in range(3):
            num = jnp.zeros((1, Q), jnp.float32)
            for a in range(-2, 3):
                for b in range(-2, 3):
                    f = c * 25 + (a + 2) * 5 + (b + 2)
                    s = a * 64 + b
                    num = num + jnp.where(masks[(a, b)],
                                          imr_scr[f:f + 1, pl.ds(256 - s, Q)],
                                          0.0)
            mean_c = misc_ref[0, 0, 1 + c]
            out_ref[0, c, :] = 127.5 * ((num[0] * rden[0]) + mean_c) + 127.5


def kernel(noisy, sigma, W1, b1, W2, b2, Wg, bg):
    t, c, h, w = noisy.shape
    pdim = c * PS * PS
    x = (noisy / 255.0 - 0.5) / 0.5
    means = x.mean(axis=(-2, -1), keepdims=True)
    x = x - means
    gw = jnp.array([0.2989, 0.587, 0.114], jnp.float32).reshape(1, 3, 1, 1)
    gray = (x * gw).sum(axis=1)
    grayp = jnp.pad(gray, ((0, 0), (PAD, PAD), (PAD, PAD)), mode='reflect')
    xp = jnp.pad(x, ((0, 0), (0, 0), (PAD, PAD), (PAD, PAD)), mode='reflect')
    g25 = jnp.stack([grayp[:, i:i + h, j:j + w].reshape(t, h * w)
                     for i in range(PS) for j in range(PS)], axis=1)
    cpsT = jnp.stack([xp[:, :, i:i + h, j:j + w].reshape(t, c, h * w)
                      for i in range(PS) for j in range(PS)],
                     axis=1).reshape(t, pdim, h * w)
    W1p = W1.reshape(c, PS * PS, -1).transpose(1, 0, 2).reshape(pdim, -1)

    sig = sigma[0] / 255.0
    T = pdim * sig * sig + 1e-6
    means_tc = means.reshape(t, c)
    misc = jnp.concatenate(
        [jnp.broadcast_to(T.reshape(1, 1), (t, 1)), means_tc,
         jnp.zeros((t, 4), jnp.float32)], axis=1).reshape(t, 1, 8)

    Wcat = jnp.concatenate([W2, Wg], axis=1)
    bcat = jnp.concatenate([b2, bg]).reshape(76, 1)
    b1r = b1.reshape(80, 1)

    grid = (t, NQB)
    out = pl.pallas_call(
        _body,
        grid=grid,
        in_specs=[
            pl.BlockSpec((1, 1, 8), lambda ti, qb: (ti, 0, 0)),
            pl.BlockSpec((1, PS * PS, h * w), lambda ti, qb: (ti, 0, 0)),
            pl.BlockSpec((1, pdim, h * w), lambda ti, qb: (ti, 0, 0)),
            pl.BlockSpec((pdim, 80), lambda ti, qb: (0, 0)),
            pl.BlockSpec((80, 1), lambda ti, qb: (0, 0)),
            pl.BlockSpec((80, 76), lambda ti, qb: (0, 0)),
            pl.BlockSpec((76, 1), lambda ti, qb: (0, 0)),
        ],
        out_specs=pl.BlockSpec((1, 3, h * w), lambda ti, qb: (ti, 0, 0)),
        out_shape=jax.ShapeDtypeStruct((t, 3, h * w), jnp.float32),
        scratch_shapes=[
            pltpu.VMEM((h * w, 80), jnp.float32),
            pltpu.VMEM((32, h * w), jnp.float32),
            pltpu.VMEM((32, h * w), jnp.float32),
            pltpu.VMEM((75, h * w + 512), jnp.float32),
            pltpu.VMEM((1, h * w + 512), jnp.float32),
        ],
    )(misc, g25, cpsT, W1p, b1r, Wcat, bcat)
    return out.reshape(t, 3, h, w)
